# Initial kernel scaffold; baseline (speedup 1.0000x reference)
import jax
import jax.numpy as jnp
from jax.experimental import pallas as pl


def kernel(x, edge_index, edge_attr, ea_w1, ea_b1, ea_w2, ea_b2, src_w, src_b, dst_w, dst_b, te_w1, te_b1, te_w2, te_b2, pd_w, pd_b, pe_w, pe_b, phi_w, phi_b):
    raise NotImplementedError("write your pallas kernel here")


# speedup vs baseline: 2.5567x; 2.5567x over previous
import functools

import jax
import jax.numpy as jnp
from jax import lax
from jax.experimental import pallas as pl
from jax.experimental.pallas import tpu as pltpu
from jax.experimental.pallas import tpu_sc as plsc

N = 10000
E = 320000
D = 128
DE = 16
H = 128

NC = 2
NS = 16
NW = NC * NS
EW = E // NW
CHUNK = 80
NCHUNK = EW // CHUNK
NP = 10240
RPT = NP // NS

_F32 = jnp.float32


def _dot(a, b):
    return jnp.dot(a, b, preferred_element_type=_F32)


def _node_pre_body(x_ref, sw_ref, sb_ref, dw_ref, db_ref, t_ref, xd_ref):
    xb = x_ref[...]
    t_ref[:, :D] = xb
    t_ref[:, D:] = _dot(xb, sw_ref[...]) + sb_ref[...]
    xd_ref[...] = _dot(xb, dw_ref[...]) + db_ref[...]


def _node_pre(x, src_w, src_b, dst_w, dst_b):
    BN = 2000
    return pl.pallas_call(
        _node_pre_body,
        grid=(N // BN,),
        in_specs=[
            pl.BlockSpec((BN, D), lambda i: (i, 0)),
            pl.BlockSpec((D, H), lambda i: (0, 0)),
            pl.BlockSpec((1, H), lambda i: (0, 0)),
            pl.BlockSpec((D, H), lambda i: (0, 0)),
            pl.BlockSpec((1, H), lambda i: (0, 0)),
        ],
        out_specs=[
            pl.BlockSpec((BN, 2 * D), lambda i: (i, 0)),
            pl.BlockSpec((BN, H), lambda i: (i, 0)),
        ],
        out_shape=[
            jax.ShapeDtypeStruct((N, 2 * D), _F32),
            jax.ShapeDtypeStruct((N, H), _F32),
        ],
    )(x, src_w, src_b, dst_w, dst_b)


_MESH = plsc.VectorSubcoreMesh(core_axis_name="c", subcore_axis_name="s")


@functools.partial(
    pl.kernel,
    mesh=_MESH,
    out_type=(
        jax.ShapeDtypeStruct((E, 2 * D), _F32),
        jax.ShapeDtypeStruct((E, D), _F32),
    ),
    scratch_types=[
        pltpu.VMEM((NCHUNK, CHUNK), jnp.int32),
        pltpu.VMEM((NCHUNK, CHUNK), jnp.int32),
        pltpu.VMEM((CHUNK, 2 * D), _F32),
        pltpu.VMEM((CHUNK, D), _F32),
    ],
)
def _gather_kernel(t_hbm, xd_hbm, sidx_hbm, didx_hbm, gt_hbm, gxd_hbm,
                   sidx_v, didx_v, rows_t, rows_d):
    c = lax.axis_index("c")
    s = lax.axis_index("s")
    wid = s * NC + c
    base = wid * EW
    pltpu.sync_copy(sidx_hbm.at[wid], sidx_v)
    pltpu.sync_copy(didx_hbm.at[wid], didx_v)

    def body(j, carry):
        off = base + j * CHUNK
        pltpu.sync_copy(t_hbm.at[sidx_v.at[j]], rows_t)
        pltpu.sync_copy(rows_t, gt_hbm.at[pl.ds(off, CHUNK)])
        pltpu.sync_copy(xd_hbm.at[didx_v.at[j]], rows_d)
        pltpu.sync_copy(rows_d, gxd_hbm.at[pl.ds(off, CHUNK)])
        return carry

    lax.fori_loop(0, NCHUNK, body, 0)


def _edge_mlp_body(gt_ref, gd_ref, ea_ref, w1_ref, b1_ref, w2_ref, b2_ref,
                   tw1_ref, tb1_ref, tw2_ref, tb2_ref, msg_ref):
    gx = gt_ref[:, :D]
    gs = gt_ref[:, D:]
    c1 = jnp.maximum(_dot(ea_ref[...], w1_ref[...]) + b1_ref[...], 0.0)
    ec = _dot(c1, w2_ref[...]) + b2_ref[...]
    s = ec + gs + gd_ref[...]
    h = jnp.maximum(_dot(jnp.maximum(s, 0.0), tw1_ref[...]) + tb1_ref[...], 0.0)
    e = _dot(h, tw2_ref[...]) + tb2_ref[...]
    msg_ref[...] = gx * e


def _edge_mlp(gt, gxd, edge_attr, ea_w1, ea_b1, ea_w2, ea_b2,
              te_w1, te_b1, te_w2, te_b2):
    BE = 1600
    full = lambda i: (0, 0)
    return pl.pallas_call(
        _edge_mlp_body,
        grid=(E // BE,),
        in_specs=[
            pl.BlockSpec((BE, 2 * D), lambda i: (i, 0)),
            pl.BlockSpec((BE, D), lambda i: (i, 0)),
            pl.BlockSpec((BE, DE), lambda i: (i, 0)),
            pl.BlockSpec((DE, H), full),
            pl.BlockSpec((1, H), full),
            pl.BlockSpec((H, H), full),
            pl.BlockSpec((1, H), full),
            pl.BlockSpec((H, H), full),
            pl.BlockSpec((1, H), full),
            pl.BlockSpec((H, D), full),
            pl.BlockSpec((1, D), full),
        ],
        out_specs=pl.BlockSpec((BE, D), lambda i: (i, 0)),
        out_shape=jax.ShapeDtypeStruct((E, D), _F32),
    )(gt, gxd, edge_attr, ea_w1, ea_b1, ea_w2, ea_b2,
      te_w1, te_b1, te_w2, te_b2)


@functools.partial(
    pl.kernel,
    mesh=_MESH,
    out_type=(
        jax.ShapeDtypeStruct((NP, D), _F32),
        jax.ShapeDtypeStruct((NP, D), _F32),
    ),
    scratch_types=[
        pltpu.VMEM((NCHUNK, CHUNK), jnp.int32),
        pltpu.VMEM((CHUNK, D), _F32),
        pltpu.VMEM_SHARED((NP, D), _F32),
    ],
)
def _scatter_kernel(msgs_hbm, didx_hbm, zeros_hbm, agg0_hbm, agg1_hbm,
                    didx_v, rows_v, acc_sh):
    c = lax.axis_index("c")
    s = lax.axis_index("s")
    wid = s * NC + c
    base = wid * EW
    pltpu.sync_copy(zeros_hbm.at[pl.ds(s * RPT, RPT)],
                    acc_sh.at[pl.ds(s * RPT, RPT)])
    pltpu.sync_copy(didx_hbm.at[wid], didx_v)
    plsc.subcore_barrier()

    def body(j, carry):
        pltpu.sync_copy(msgs_hbm.at[pl.ds(base + j * CHUNK, CHUNK)], rows_v)
        pltpu.sync_copy(rows_v, acc_sh.at[didx_v.at[j]], add=True)
        return carry

    lax.fori_loop(0, NCHUNK, body, 0)
    plsc.subcore_barrier()

    @pl.when(c == 0)
    def _():
        pltpu.sync_copy(acc_sh.at[pl.ds(s * RPT, RPT)],
                        agg0_hbm.at[pl.ds(s * RPT, RPT)])

    @pl.when(c == 1)
    def _():
        pltpu.sync_copy(acc_sh.at[pl.ds(s * RPT, RPT)],
                        agg1_hbm.at[pl.ds(s * RPT, RPT)])


def _final_body(x_ref, p0_ref, p1_ref, pdw_ref, pdb_ref, pew_ref, peb_ref,
                phw_ref, phb_ref, out_ref):
    agg = p0_ref[...] + p1_ref[...]
    pre = _dot(x_ref[...], pdw_ref[...]) + pdb_ref[...]
    pre = pre + _dot(agg, pew_ref[...]) + peb_ref[...]
    out_ref[...] = _dot(jnp.maximum(pre, 0.0), phw_ref[...]) + phb_ref[...]


def _final(x, agg0, agg1, pd_w, pd_b, pe_w, pe_b, phi_w, phi_b):
    BN = 2000
    full = lambda i: (0, 0)
    nb = N // BN
    return pl.pallas_call(
        _final_body,
        grid=(nb,),
        in_specs=[
            pl.BlockSpec((BN, D), lambda i: (i, 0)),
            pl.BlockSpec((BN, D), lambda i: (i, 0)),
            pl.BlockSpec((BN, D), lambda i: (i, 0)),
            pl.BlockSpec((D, H), full),
            pl.BlockSpec((1, H), full),
            pl.BlockSpec((D, H), full),
            pl.BlockSpec((1, H), full),
            pl.BlockSpec((H, D), full),
            pl.BlockSpec((1, D), full),
        ],
        out_specs=pl.BlockSpec((BN, D), lambda i: (i, 0)),
        out_shape=jax.ShapeDtypeStruct((N, D), _F32),
    )(x, agg0, agg1, pd_w, pd_b, pe_w, pe_b, phi_w, phi_b)


def kernel(x, edge_index, edge_attr, ea_w1, ea_b1, ea_w2, ea_b2,
           src_w, src_b, dst_w, dst_b, te_w1, te_b1, te_w2, te_b2,
           pd_w, pd_b, pe_w, pe_b, phi_w, phi_b):
    sidx = edge_index[0].reshape(NW, NCHUNK, CHUNK)
    didx = edge_index[1].reshape(NW, NCHUNK, CHUNK)

    r = lambda b: b.reshape(1, -1)

    t, xd = _node_pre(x, src_w, r(src_b), dst_w, r(dst_b))
    gt, gxd = _gather_kernel(t, xd, sidx, didx)
    msgs = _edge_mlp(gt, gxd, edge_attr, ea_w1, r(ea_b1), ea_w2, r(ea_b2),
                     te_w1, r(te_b1), te_w2, r(te_b2))
    zeros = jnp.zeros((NP, D), _F32)
    agg0, agg1 = _scatter_kernel(msgs, didx, zeros)
    return _final(x, agg0, agg1, pd_w, r(pd_b), pe_w, r(pe_b), phi_w, r(phi_b))


# speedup vs baseline: 2.9108x; 1.1385x over previous
import functools

import jax
import jax.numpy as jnp
from jax import lax
from jax.experimental import pallas as pl
from jax.experimental.pallas import tpu as pltpu
from jax.experimental.pallas import tpu_sc as plsc

N = 10000
E = 320000
D = 128
DE = 16
H = 128

NC = 2
NS = 16
NW = NC * NS
EW = E // NW
CHUNK = 80
NCHUNK = EW // CHUNK
NP = 10240
RPT = NP // NS

_F32 = jnp.float32
_BF16 = jnp.bfloat16


def _dot(a, b):
    return jnp.dot(a, b, preferred_element_type=_F32)


def _bits16(v):
    return jax.lax.bitcast_convert_type(v.astype(_BF16), jnp.uint16).astype(jnp.uint32)


def _pack2(lo, hi):
    return jax.lax.bitcast_convert_type(_bits16(lo) | (_bits16(hi) << 16), _F32)


def _unpack_lo(w):
    return jax.lax.bitcast_convert_type(w << 16, _F32)


def _unpack_hi(w):
    return jax.lax.bitcast_convert_type(w & jnp.uint32(0xFFFF0000), _F32)


def _node_pre_body(x_ref, sw_ref, sb_ref, dw_ref, db_ref, t_ref, xd_ref):
    xb = x_ref[...]
    xs = _dot(xb, sw_ref[...]) + sb_ref[...]
    xd = _dot(xb, dw_ref[...]) + db_ref[...]
    t_ref[...] = _pack2(xb, xs)
    xd_ref[...] = xd


def _node_pre(x, src_w, src_b, dst_w, dst_b):
    BN = 2000
    return pl.pallas_call(
        _node_pre_body,
        grid=(N // BN,),
        in_specs=[
            pl.BlockSpec((BN, D), lambda i: (i, 0)),
            pl.BlockSpec((D, H), lambda i: (0, 0)),
            pl.BlockSpec((1, H), lambda i: (0, 0)),
            pl.BlockSpec((D, H), lambda i: (0, 0)),
            pl.BlockSpec((1, H), lambda i: (0, 0)),
        ],
        out_specs=[
            pl.BlockSpec((BN, D), lambda i: (i, 0)),
            pl.BlockSpec((BN, H), lambda i: (i, 0)),
        ],
        out_shape=[
            jax.ShapeDtypeStruct((N, D), _F32),
            jax.ShapeDtypeStruct((N, H), _F32),
        ],
    )(x, src_w, src_b, dst_w, dst_b)


_MESH = plsc.VectorSubcoreMesh(core_axis_name="c", subcore_axis_name="s")


@functools.partial(
    pl.kernel,
    mesh=_MESH,
    out_type=(
        jax.ShapeDtypeStruct((E, D), _F32),
        jax.ShapeDtypeStruct((E, D), _F32),
    ),
    scratch_types=[
        pltpu.VMEM((NCHUNK, CHUNK), jnp.int32),
        pltpu.VMEM((NCHUNK, CHUNK), jnp.int32),
        pltpu.VMEM((CHUNK, D), _F32),
        pltpu.VMEM((CHUNK, D), _F32),
    ],
)
def _gather_kernel(t_hbm, xd_hbm, sidx_hbm, didx_hbm, gt_hbm, gxd_hbm,
                   sidx_v, didx_v, rows_t, rows_d):
    c = lax.axis_index("c")
    s = lax.axis_index("s")
    wid = s * NC + c
    base = wid * EW
    pltpu.sync_copy(sidx_hbm.at[wid], sidx_v)
    pltpu.sync_copy(didx_hbm.at[wid], didx_v)

    def body(j, carry):
        off = base + j * CHUNK
        pltpu.sync_copy(t_hbm.at[sidx_v.at[j]], rows_t)
        pltpu.sync_copy(rows_t, gt_hbm.at[pl.ds(off, CHUNK)])
        pltpu.sync_copy(xd_hbm.at[didx_v.at[j]], rows_d)
        pltpu.sync_copy(rows_d, gxd_hbm.at[pl.ds(off, CHUNK)])
        return carry

    lax.fori_loop(0, NCHUNK, body, 0)


def _edge_mlp_body(gt_ref, gd_ref, ea_ref, w1_ref, b1_ref, w2_ref, b2_ref,
                   tw1_ref, tb1_ref, tw2_ref, tb2_ref, msg_ref):
    wt = jax.lax.bitcast_convert_type(gt_ref[...], jnp.uint32)
    gx = _unpack_lo(wt)
    gs = _unpack_hi(wt)
    c1 = jnp.maximum(_dot(ea_ref[...], w1_ref[...]) + b1_ref[...], 0.0)
    ec = _dot(c1, w2_ref[...]) + b2_ref[...]
    s = ec + gs + gd_ref[...]
    h = jnp.maximum(_dot(jnp.maximum(s, 0.0), tw1_ref[...]) + tb1_ref[...], 0.0)
    e = _dot(h, tw2_ref[...]) + tb2_ref[...]
    msg_ref[...] = gx * e


def _edge_mlp(gt, gxd, edge_attr, ea_w1, ea_b1, ea_w2, ea_b2,
              te_w1, te_b1, te_w2, te_b2):
    BE = 1600
    full = lambda i: (0, 0)
    return pl.pallas_call(
        _edge_mlp_body,
        grid=(E // BE,),
        in_specs=[
            pl.BlockSpec((BE, D), lambda i: (i, 0)),
            pl.BlockSpec((BE, D), lambda i: (i, 0)),
            pl.BlockSpec((BE, DE), lambda i: (i, 0)),
            pl.BlockSpec((DE, H), full),
            pl.BlockSpec((1, H), full),
            pl.BlockSpec((H, H), full),
            pl.BlockSpec((1, H), full),
            pl.BlockSpec((H, H), full),
            pl.BlockSpec((1, H), full),
            pl.BlockSpec((H, D), full),
            pl.BlockSpec((1, D), full),
        ],
        out_specs=pl.BlockSpec((BE, D), lambda i: (i, 0)),
        out_shape=jax.ShapeDtypeStruct((E, D), _F32),
    )(gt, gxd, edge_attr, ea_w1, ea_b1, ea_w2, ea_b2,
      te_w1, te_b1, te_w2, te_b2)


@functools.partial(
    pl.kernel,
    mesh=_MESH,
    out_type=(
        jax.ShapeDtypeStruct((NP, D), _F32),
        jax.ShapeDtypeStruct((NP, D), _F32),
    ),
    scratch_types=[
        pltpu.VMEM((NCHUNK, CHUNK), jnp.int32),
        pltpu.VMEM((CHUNK, D), _F32),
        pltpu.VMEM_SHARED((NP, D), _F32),
    ],
)
def _scatter_kernel(msgs_hbm, didx_hbm, zeros_hbm, agg0_hbm, agg1_hbm,
                    didx_v, rows_v, acc_sh):
    c = lax.axis_index("c")
    s = lax.axis_index("s")
    wid = s * NC + c
    base = wid * EW
    pltpu.sync_copy(zeros_hbm.at[pl.ds(s * RPT, RPT)],
                    acc_sh.at[pl.ds(s * RPT, RPT)])
    pltpu.sync_copy(didx_hbm.at[wid], didx_v)
    plsc.subcore_barrier()

    def body(j, carry):
        pltpu.sync_copy(msgs_hbm.at[pl.ds(base + j * CHUNK, CHUNK)], rows_v)
        pltpu.sync_copy(rows_v, acc_sh.at[didx_v.at[j]], add=True)
        return carry

    lax.fori_loop(0, NCHUNK, body, 0)
    plsc.subcore_barrier()

    @pl.when(c == 0)
    def _():
        pltpu.sync_copy(acc_sh.at[pl.ds(s * RPT, RPT)],
                        agg0_hbm.at[pl.ds(s * RPT, RPT)])

    @pl.when(c == 1)
    def _():
        pltpu.sync_copy(acc_sh.at[pl.ds(s * RPT, RPT)],
                        agg1_hbm.at[pl.ds(s * RPT, RPT)])


def _final_body(x_ref, p0_ref, p1_ref, pdw_ref, pdb_ref, pew_ref, peb_ref,
                phw_ref, phb_ref, out_ref):
    agg = p0_ref[...] + p1_ref[...]
    pre = _dot(x_ref[...], pdw_ref[...]) + pdb_ref[...]
    pre = pre + _dot(agg, pew_ref[...]) + peb_ref[...]
    out_ref[...] = _dot(jnp.maximum(pre, 0.0), phw_ref[...]) + phb_ref[...]


def _final(x, agg0, agg1, pd_w, pd_b, pe_w, pe_b, phi_w, phi_b):
    BN = 2000
    full = lambda i: (0, 0)
    nb = N // BN
    return pl.pallas_call(
        _final_body,
        grid=(nb,),
        in_specs=[
            pl.BlockSpec((BN, D), lambda i: (i, 0)),
            pl.BlockSpec((BN, D), lambda i: (i, 0)),
            pl.BlockSpec((BN, D), lambda i: (i, 0)),
            pl.BlockSpec((D, H), full),
            pl.BlockSpec((1, H), full),
            pl.BlockSpec((D, H), full),
            pl.BlockSpec((1, H), full),
            pl.BlockSpec((H, D), full),
            pl.BlockSpec((1, D), full),
        ],
        out_specs=pl.BlockSpec((BN, D), lambda i: (i, 0)),
        out_shape=jax.ShapeDtypeStruct((N, D), _F32),
    )(x, agg0, agg1, pd_w, pd_b, pe_w, pe_b, phi_w, phi_b)


def kernel(x, edge_index, edge_attr, ea_w1, ea_b1, ea_w2, ea_b2,
           src_w, src_b, dst_w, dst_b, te_w1, te_b1, te_w2, te_b2,
           pd_w, pd_b, pe_w, pe_b, phi_w, phi_b):
    sidx = edge_index[0].reshape(NW, NCHUNK, CHUNK)
    didx = edge_index[1].reshape(NW, NCHUNK, CHUNK)

    r = lambda b: b.reshape(1, -1)

    t, xd = _node_pre(x, src_w, r(src_b), dst_w, r(dst_b))
    gt, gxd = _gather_kernel(t, xd, sidx, didx)
    msgs = _edge_mlp(gt, gxd, edge_attr, ea_w1, r(ea_b1), ea_w2, r(ea_b2),
                     te_w1, r(te_b1), te_w2, r(te_b2))
    zeros = jnp.zeros((NP, D), _F32)
    agg0, agg1 = _scatter_kernel(msgs, didx, zeros)
    return _final(x, agg0, agg1, pd_w, r(pd_b), pe_w, r(pe_b), phi_w, r(phi_b))


# speedup vs baseline: 3.5910x; 1.2337x over previous
import functools

import jax
import jax.numpy as jnp
from jax import lax
from jax.experimental import pallas as pl
from jax.experimental.pallas import tpu as pltpu
from jax.experimental.pallas import tpu_sc as plsc

N = 10000
E = 320000
D = 128
DE = 16
H = 128

NC = 2
NS = 16
NW = NC * NS
EW = E // NW
CHUNK = 80
NCHUNK = EW // CHUNK
NP = 10240
RPT = NP // NS

_F32 = jnp.float32
_BF16 = jnp.bfloat16


def _dot(a, b):
    return jnp.dot(a, b, preferred_element_type=_F32)


def _bits16(v):
    return jax.lax.bitcast_convert_type(v.astype(_BF16), jnp.uint16).astype(jnp.uint32)


def _pack2(lo, hi):
    return jax.lax.bitcast_convert_type(_bits16(lo) | (_bits16(hi) << 16), _F32)


def _unpack_lo(w):
    return jax.lax.bitcast_convert_type(w << 16, _F32)


def _unpack_hi(w):
    return jax.lax.bitcast_convert_type(w & jnp.uint32(0xFFFF0000), _F32)


def _node_pre_body(x_ref, sw_ref, sb_ref, dw_ref, db_ref, t_ref, xd_ref):
    xb = x_ref[...]
    xs = _dot(xb, sw_ref[...]) + sb_ref[...]
    xd = _dot(xb, dw_ref[...]) + db_ref[...]
    t_ref[...] = _pack2(xb, xs)
    xd_ref[...] = xd


def _node_pre(x, src_w, src_b, dst_w, dst_b):
    BN = 2000
    return pl.pallas_call(
        _node_pre_body,
        grid=(N // BN,),
        in_specs=[
            pl.BlockSpec((BN, D), lambda i: (i, 0)),
            pl.BlockSpec((D, H), lambda i: (0, 0)),
            pl.BlockSpec((1, H), lambda i: (0, 0)),
            pl.BlockSpec((D, H), lambda i: (0, 0)),
            pl.BlockSpec((1, H), lambda i: (0, 0)),
        ],
        out_specs=[
            pl.BlockSpec((BN, D), lambda i: (i, 0)),
            pl.BlockSpec((BN, H), lambda i: (i, 0)),
        ],
        out_shape=[
            jax.ShapeDtypeStruct((N, D), _F32),
            jax.ShapeDtypeStruct((N, H), _F32),
        ],
    )(x, src_w, src_b, dst_w, dst_b)


_MESH = plsc.VectorSubcoreMesh(core_axis_name="c", subcore_axis_name="s")


@functools.partial(
    pl.kernel,
    mesh=_MESH,
    out_type=(
        jax.ShapeDtypeStruct((E, D), _F32),
        jax.ShapeDtypeStruct((E, D), _F32),
    ),
    scratch_types=[
        pltpu.VMEM((NCHUNK, CHUNK), jnp.int32),
        pltpu.VMEM((NCHUNK, CHUNK), jnp.int32),
        pltpu.VMEM((CHUNK, D), _F32),
        pltpu.VMEM((CHUNK, D), _F32),
        pltpu.VMEM((CHUNK, D), _F32),
        pltpu.VMEM((CHUNK, D), _F32),
        pltpu.SemaphoreType.DMA,
        pltpu.SemaphoreType.DMA,
    ],
)
def _gather_kernel(t_hbm, xd_hbm, sidx_hbm, didx_hbm, gt_hbm, gxd_hbm,
                   sidx_v, didx_v, rt0, rt1, rd0, rd1, sg0, sg1):
    c = lax.axis_index("c")
    s = lax.axis_index("s")
    wid = s * NC + c
    base = wid * EW
    pltpu.sync_copy(sidx_hbm.at[wid], sidx_v)
    pltpu.sync_copy(didx_hbm.at[wid], didx_v)
    rts = (rt0, rt1)
    rds = (rd0, rd1)
    sgs = (sg0, sg1)

    def issue(j, b):
        pltpu.async_copy(t_hbm.at[sidx_v.at[j]], rts[b], sgs[b])
        pltpu.async_copy(xd_hbm.at[didx_v.at[j]], rds[b], sgs[b])

    def wait(j, b):
        pltpu.make_async_copy(t_hbm.at[sidx_v.at[j]], rts[b], sgs[b]).wait()
        pltpu.make_async_copy(xd_hbm.at[didx_v.at[j]], rds[b], sgs[b]).wait()

    def emit(j, b):
        off = base + j * CHUNK
        pltpu.sync_copy(rts[b], gt_hbm.at[pl.ds(off, CHUNK)])
        pltpu.sync_copy(rds[b], gxd_hbm.at[pl.ds(off, CHUNK)])

    issue(0, 0)

    def body(jj, carry):
        for b in range(2):
            j = 2 * jj + b
            wait(j, b)
            issue(j + 1, 1 - b)
            emit(j, b)
        return carry

    lax.fori_loop(0, (NCHUNK - 1) // 2, body, 0)
    wait(NCHUNK - 1, (NCHUNK - 1) % 2)
    emit(NCHUNK - 1, (NCHUNK - 1) % 2)


def _edge_mlp_body(gt_ref, gd_ref, ea_ref, w1_ref, b1_ref, w2_ref, b2_ref,
                   tw1_ref, tb1_ref, tw2_ref, tb2_ref, msg_ref):
    wt = jax.lax.bitcast_convert_type(gt_ref[...], jnp.uint32)
    gx = _unpack_lo(wt)
    gs = _unpack_hi(wt)
    c1 = jnp.maximum(_dot(ea_ref[...], w1_ref[...]) + b1_ref[...], 0.0)
    ec = _dot(c1, w2_ref[...]) + b2_ref[...]
    s = ec + gs + gd_ref[...]
    h = jnp.maximum(_dot(jnp.maximum(s, 0.0), tw1_ref[...]) + tb1_ref[...], 0.0)
    e = _dot(h, tw2_ref[...]) + tb2_ref[...]
    msg_ref[...] = gx * e


def _edge_mlp(gt, gxd, edge_attr, ea_w1, ea_b1, ea_w2, ea_b2,
              te_w1, te_b1, te_w2, te_b2):
    BE = 1600
    full = lambda i: (0, 0)
    return pl.pallas_call(
        _edge_mlp_body,
        grid=(E // BE,),
        in_specs=[
            pl.BlockSpec((BE, D), lambda i: (i, 0)),
            pl.BlockSpec((BE, D), lambda i: (i, 0)),
            pl.BlockSpec((BE, DE), lambda i: (i, 0)),
            pl.BlockSpec((DE, H), full),
            pl.BlockSpec((1, H), full),
            pl.BlockSpec((H, H), full),
            pl.BlockSpec((1, H), full),
            pl.BlockSpec((H, H), full),
            pl.BlockSpec((1, H), full),
            pl.BlockSpec((H, D), full),
            pl.BlockSpec((1, D), full),
        ],
        out_specs=pl.BlockSpec((BE, D), lambda i: (i, 0)),
        out_shape=jax.ShapeDtypeStruct((E, D), _F32),
    )(gt, gxd, edge_attr, ea_w1, ea_b1, ea_w2, ea_b2,
      te_w1, te_b1, te_w2, te_b2)


@functools.partial(
    pl.kernel,
    mesh=_MESH,
    out_type=(
        jax.ShapeDtypeStruct((NP, D), _F32),
        jax.ShapeDtypeStruct((NP, D), _F32),
    ),
    scratch_types=[
        pltpu.VMEM((NCHUNK, CHUNK), jnp.int32),
        pltpu.VMEM((CHUNK, D), _F32),
        pltpu.VMEM((CHUNK, D), _F32),
        pltpu.VMEM_SHARED((NP, D), _F32),
        pltpu.SemaphoreType.DMA,
        pltpu.SemaphoreType.DMA,
    ],
)
def _scatter_kernel(msgs_hbm, didx_hbm, zeros_hbm, agg0_hbm, agg1_hbm,
                    didx_v, rv0, rv1, acc_sh, sm0, sm1):
    c = lax.axis_index("c")
    s = lax.axis_index("s")
    wid = s * NC + c
    base = wid * EW
    pltpu.sync_copy(zeros_hbm.at[pl.ds(s * RPT, RPT)],
                    acc_sh.at[pl.ds(s * RPT, RPT)])
    pltpu.sync_copy(didx_hbm.at[wid], didx_v)
    plsc.subcore_barrier()
    rvs = (rv0, rv1)
    sms = (sm0, sm1)

    def issue(j, b):
        pltpu.async_copy(msgs_hbm.at[pl.ds(base + j * CHUNK, CHUNK)],
                         rvs[b], sms[b])

    def wait(j, b):
        pltpu.make_async_copy(msgs_hbm.at[pl.ds(base + j * CHUNK, CHUNK)],
                              rvs[b], sms[b]).wait()

    issue(0, 0)

    def body(jj, carry):
        for b in range(2):
            j = 2 * jj + b
            wait(j, b)
            issue(j + 1, 1 - b)
            pltpu.sync_copy(rvs[b], acc_sh.at[didx_v.at[j]], add=True)
        return carry

    lax.fori_loop(0, (NCHUNK - 1) // 2, body, 0)
    jl = NCHUNK - 1
    wait(jl, jl % 2)
    pltpu.sync_copy(rvs[jl % 2], acc_sh.at[didx_v.at[jl]], add=True)
    plsc.subcore_barrier()

    @pl.when(c == 0)
    def _():
        pltpu.sync_copy(acc_sh.at[pl.ds(s * RPT, RPT)],
                        agg0_hbm.at[pl.ds(s * RPT, RPT)])

    @pl.when(c == 1)
    def _():
        pltpu.sync_copy(acc_sh.at[pl.ds(s * RPT, RPT)],
                        agg1_hbm.at[pl.ds(s * RPT, RPT)])


def _final_body(x_ref, p0_ref, p1_ref, pdw_ref, pdb_ref, pew_ref, peb_ref,
                phw_ref, phb_ref, out_ref):
    agg = p0_ref[...] + p1_ref[...]
    pre = _dot(x_ref[...], pdw_ref[...]) + pdb_ref[...]
    pre = pre + _dot(agg, pew_ref[...]) + peb_ref[...]
    out_ref[...] = _dot(jnp.maximum(pre, 0.0), phw_ref[...]) + phb_ref[...]


def _final(x, agg0, agg1, pd_w, pd_b, pe_w, pe_b, phi_w, phi_b):
    BN = 2000
    full = lambda i: (0, 0)
    nb = N // BN
    return pl.pallas_call(
        _final_body,
        grid=(nb,),
        in_specs=[
            pl.BlockSpec((BN, D), lambda i: (i, 0)),
            pl.BlockSpec((BN, D), lambda i: (i, 0)),
            pl.BlockSpec((BN, D), lambda i: (i, 0)),
            pl.BlockSpec((D, H), full),
            pl.BlockSpec((1, H), full),
            pl.BlockSpec((D, H), full),
            pl.BlockSpec((1, H), full),
            pl.BlockSpec((H, D), full),
            pl.BlockSpec((1, D), full),
        ],
        out_specs=pl.BlockSpec((BN, D), lambda i: (i, 0)),
        out_shape=jax.ShapeDtypeStruct((N, D), _F32),
    )(x, agg0, agg1, pd_w, pd_b, pe_w, pe_b, phi_w, phi_b)


def kernel(x, edge_index, edge_attr, ea_w1, ea_b1, ea_w2, ea_b2,
           src_w, src_b, dst_w, dst_b, te_w1, te_b1, te_w2, te_b2,
           pd_w, pd_b, pe_w, pe_b, phi_w, phi_b):
    sidx = edge_index[0].reshape(NW, NCHUNK, CHUNK)
    didx = edge_index[1].reshape(NW, NCHUNK, CHUNK)

    r = lambda b: b.reshape(1, -1)

    t, xd = _node_pre(x, src_w, r(src_b), dst_w, r(dst_b))
    gt, gxd = _gather_kernel(t, xd, sidx, didx)
    msgs = _edge_mlp(gt, gxd, edge_attr, ea_w1, r(ea_b1), ea_w2, r(ea_b2),
                     te_w1, r(te_b1), te_w2, r(te_b2))
    zeros = jnp.zeros((NP, D), _F32)
    agg0, agg1 = _scatter_kernel(msgs, didx, zeros)
    return _final(x, agg0, agg1, pd_w, r(pd_b), pe_w, r(pe_b), phi_w, r(phi_b))


# speedup vs baseline: 3.6161x; 1.0070x over previous
import functools

import jax
import jax.numpy as jnp
from jax import lax
from jax.experimental import pallas as pl
from jax.experimental.pallas import tpu as pltpu
from jax.experimental.pallas import tpu_sc as plsc

N = 10000
E = 320000
D = 128
DE = 16
H = 128

NC = 2
NS = 16
NW = NC * NS
NSPLIT = 2
ES = E // NSPLIT
EWS = ES // NW
CHUNK = 40
NCHUNK = EWS // CHUNK
NP = 10240
RPT = NP // NS

_F32 = jnp.float32
_BF16 = jnp.bfloat16


def _dot(a, b):
    return jnp.dot(a, b, preferred_element_type=_F32)


def _bits16(v):
    return jax.lax.bitcast_convert_type(v.astype(_BF16), jnp.uint16).astype(jnp.uint32)


def _pack2(lo, hi):
    return jax.lax.bitcast_convert_type(_bits16(lo) | (_bits16(hi) << 16), _F32)


def _unpack_lo(w):
    return jax.lax.bitcast_convert_type(w << 16, _F32)


def _unpack_hi(w):
    return jax.lax.bitcast_convert_type(w & jnp.uint32(0xFFFF0000), _F32)


def _node_pre_body(x_ref, sw_ref, sb_ref, dw_ref, db_ref, t_ref, xd_ref):
    xb = x_ref[...]
    xs = _dot(xb, sw_ref[...]) + sb_ref[...]
    xd = _dot(xb, dw_ref[...]) + db_ref[...]
    t_ref[...] = _pack2(xb, xs)
    xd_ref[...] = xd


def _node_pre(x, src_w, src_b, dst_w, dst_b):
    BN = 2000
    return pl.pallas_call(
        _node_pre_body,
        grid=(N // BN,),
        in_specs=[
            pl.BlockSpec((BN, D), lambda i: (i, 0)),
            pl.BlockSpec((D, H), lambda i: (0, 0)),
            pl.BlockSpec((1, H), lambda i: (0, 0)),
            pl.BlockSpec((D, H), lambda i: (0, 0)),
            pl.BlockSpec((1, H), lambda i: (0, 0)),
        ],
        out_specs=[
            pl.BlockSpec((BN, D), lambda i: (i, 0)),
            pl.BlockSpec((BN, H), lambda i: (i, 0)),
        ],
        out_shape=[
            jax.ShapeDtypeStruct((N, D), _F32),
            jax.ShapeDtypeStruct((N, H), _F32),
        ],
    )(x, src_w, src_b, dst_w, dst_b)


_MESH = plsc.VectorSubcoreMesh(core_axis_name="c", subcore_axis_name="s")


@functools.partial(
    pl.kernel,
    mesh=_MESH,
    out_type=(
        jax.ShapeDtypeStruct((ES, D), _F32),
        jax.ShapeDtypeStruct((ES, D), _F32),
    ),
    scratch_types=[
        pltpu.VMEM((NCHUNK, CHUNK), jnp.int32),
        pltpu.VMEM((NCHUNK, CHUNK), jnp.int32),
        pltpu.VMEM((CHUNK, D), _F32),
        pltpu.VMEM((CHUNK, D), _F32),
        pltpu.VMEM((CHUNK, D), _F32),
        pltpu.VMEM((CHUNK, D), _F32),
        pltpu.SemaphoreType.DMA,
        pltpu.SemaphoreType.DMA,
    ],
)
def _gather_kernel(t_hbm, xd_hbm, sidx_hbm, didx_hbm, gt_hbm, gxd_hbm,
                   sidx_v, didx_v, rt0, rt1, rd0, rd1, sg0, sg1):
    c = lax.axis_index("c")
    s = lax.axis_index("s")
    wid = s * NC + c
    base = wid * EWS
    pltpu.sync_copy(sidx_hbm.at[wid], sidx_v)
    pltpu.sync_copy(didx_hbm.at[wid], didx_v)
    rts = (rt0, rt1)
    rds = (rd0, rd1)
    sgs = (sg0, sg1)

    def issue(j, b):
        pltpu.async_copy(t_hbm.at[sidx_v.at[j]], rts[b], sgs[b])
        pltpu.async_copy(xd_hbm.at[didx_v.at[j]], rds[b], sgs[b])

    def wait(j, b):
        pltpu.make_async_copy(t_hbm.at[sidx_v.at[j]], rts[b], sgs[b]).wait()
        pltpu.make_async_copy(xd_hbm.at[didx_v.at[j]], rds[b], sgs[b]).wait()

    def emit(j, b):
        off = base + j * CHUNK
        pltpu.sync_copy(rts[b], gt_hbm.at[pl.ds(off, CHUNK)])
        pltpu.sync_copy(rds[b], gxd_hbm.at[pl.ds(off, CHUNK)])

    issue(0, 0)

    def body(jj, carry):
        for b in range(2):
            j = 2 * jj + b
            wait(j, b)
            issue(j + 1, 1 - b)
            emit(j, b)
        return carry

    lax.fori_loop(0, (NCHUNK - 1) // 2, body, 0)
    wait(NCHUNK - 1, (NCHUNK - 1) % 2)
    emit(NCHUNK - 1, (NCHUNK - 1) % 2)


def _edge_mlp_body(gt_ref, gd_ref, ea_ref, w1_ref, b1_ref, w2_ref, b2_ref,
                   tw1_ref, tb1_ref, tw2_ref, tb2_ref, msg_ref):
    wt = jax.lax.bitcast_convert_type(gt_ref[...], jnp.uint32)
    gx = _unpack_lo(wt)
    gs = _unpack_hi(wt)
    c1 = jnp.maximum(_dot(ea_ref[...], w1_ref[...]) + b1_ref[...], 0.0)
    ec = _dot(c1, w2_ref[...]) + b2_ref[...]
    s = ec + gs + gd_ref[...]
    h = jnp.maximum(_dot(jnp.maximum(s, 0.0), tw1_ref[...]) + tb1_ref[...], 0.0)
    e = _dot(h, tw2_ref[...]) + tb2_ref[...]
    msg_ref[...] = gx * e


def _edge_mlp(half, gt, gxd, edge_attr, ea_w1, ea_b1, ea_w2, ea_b2,
              te_w1, te_b1, te_w2, te_b2):
    BE = 1600
    off = half * (ES // BE)
    full = lambda i: (0, 0)
    return pl.pallas_call(
        _edge_mlp_body,
        grid=(ES // BE,),
        in_specs=[
            pl.BlockSpec((BE, D), lambda i: (i, 0)),
            pl.BlockSpec((BE, D), lambda i: (i, 0)),
            pl.BlockSpec((BE, DE), lambda i: (i + off, 0)),
            pl.BlockSpec((DE, H), full),
            pl.BlockSpec((1, H), full),
            pl.BlockSpec((H, H), full),
            pl.BlockSpec((1, H), full),
            pl.BlockSpec((H, H), full),
            pl.BlockSpec((1, H), full),
            pl.BlockSpec((H, D), full),
            pl.BlockSpec((1, D), full),
        ],
        out_specs=pl.BlockSpec((BE, D), lambda i: (i, 0)),
        out_shape=jax.ShapeDtypeStruct((ES, D), _F32),
    )(gt, gxd, edge_attr, ea_w1, ea_b1, ea_w2, ea_b2,
      te_w1, te_b1, te_w2, te_b2)


@functools.partial(
    pl.kernel,
    mesh=_MESH,
    out_type=(
        jax.ShapeDtypeStruct((NP, D), _F32),
        jax.ShapeDtypeStruct((NP, D), _F32),
    ),
    scratch_types=[
        pltpu.VMEM((NCHUNK, CHUNK), jnp.int32),
        pltpu.VMEM((CHUNK, D), _F32),
        pltpu.VMEM((CHUNK, D), _F32),
        pltpu.VMEM_SHARED((NP, D), _F32),
        pltpu.SemaphoreType.DMA,
        pltpu.SemaphoreType.DMA,
    ],
)
def _scatter_kernel(msgs_hbm, didx_hbm, zeros_hbm, agg0_hbm, agg1_hbm,
                    didx_v, rv0, rv1, acc_sh, sm0, sm1):
    c = lax.axis_index("c")
    s = lax.axis_index("s")
    wid = s * NC + c
    base = wid * EWS
    pltpu.sync_copy(zeros_hbm.at[pl.ds(s * RPT, RPT)],
                    acc_sh.at[pl.ds(s * RPT, RPT)])
    pltpu.sync_copy(didx_hbm.at[wid], didx_v)
    plsc.subcore_barrier()
    rvs = (rv0, rv1)
    sms = (sm0, sm1)

    def issue(j, b):
        pltpu.async_copy(msgs_hbm.at[pl.ds(base + j * CHUNK, CHUNK)],
                         rvs[b], sms[b])

    def wait(j, b):
        pltpu.make_async_copy(msgs_hbm.at[pl.ds(base + j * CHUNK, CHUNK)],
                              rvs[b], sms[b]).wait()

    issue(0, 0)

    def body(jj, carry):
        for b in range(2):
            j = 2 * jj + b
            wait(j, b)
            issue(j + 1, 1 - b)
            pltpu.sync_copy(rvs[b], acc_sh.at[didx_v.at[j]], add=True)
        return carry

    lax.fori_loop(0, (NCHUNK - 1) // 2, body, 0)
    jl = NCHUNK - 1
    wait(jl, jl % 2)
    pltpu.sync_copy(rvs[jl % 2], acc_sh.at[didx_v.at[jl]], add=True)
    plsc.subcore_barrier()

    @pl.when(c == 0)
    def _():
        pltpu.sync_copy(acc_sh.at[pl.ds(s * RPT, RPT)],
                        agg0_hbm.at[pl.ds(s * RPT, RPT)])

    @pl.when(c == 1)
    def _():
        pltpu.sync_copy(acc_sh.at[pl.ds(s * RPT, RPT)],
                        agg1_hbm.at[pl.ds(s * RPT, RPT)])


def _final_body(x_ref, p0_ref, p1_ref, p2_ref, p3_ref, pdw_ref, pdb_ref,
                pew_ref, peb_ref, phw_ref, phb_ref, out_ref):
    agg = p0_ref[...] + p1_ref[...] + p2_ref[...] + p3_ref[...]
    pre = _dot(x_ref[...], pdw_ref[...]) + pdb_ref[...]
    pre = pre + _dot(agg, pew_ref[...]) + peb_ref[...]
    out_ref[...] = _dot(jnp.maximum(pre, 0.0), phw_ref[...]) + phb_ref[...]


def _final(x, aggs, pd_w, pd_b, pe_w, pe_b, phi_w, phi_b):
    BN = 2000
    full = lambda i: (0, 0)
    blk = lambda i: (i, 0)
    return pl.pallas_call(
        _final_body,
        grid=(N // BN,),
        in_specs=[
            pl.BlockSpec((BN, D), blk),
            pl.BlockSpec((BN, D), blk),
            pl.BlockSpec((BN, D), blk),
            pl.BlockSpec((BN, D), blk),
            pl.BlockSpec((BN, D), blk),
            pl.BlockSpec((D, H), full),
            pl.BlockSpec((1, H), full),
            pl.BlockSpec((D, H), full),
            pl.BlockSpec((1, H), full),
            pl.BlockSpec((H, D), full),
            pl.BlockSpec((1, D), full),
        ],
        out_specs=pl.BlockSpec((BN, D), blk),
        out_shape=jax.ShapeDtypeStruct((N, D), _F32),
    )(x, *aggs, pd_w, pd_b, pe_w, pe_b, phi_w, phi_b)


def kernel(x, edge_index, edge_attr, ea_w1, ea_b1, ea_w2, ea_b2,
           src_w, src_b, dst_w, dst_b, te_w1, te_b1, te_w2, te_b2,
           pd_w, pd_b, pe_w, pe_b, phi_w, phi_b):
    sidx = edge_index[0].reshape(NSPLIT, NW, NCHUNK, CHUNK)
    didx = edge_index[1].reshape(NSPLIT, NW, NCHUNK, CHUNK)

    r = lambda b: b.reshape(1, -1)

    t, xd = _node_pre(x, src_w, r(src_b), dst_w, r(dst_b))
    zeros = jnp.zeros((NP, D), _F32)

    aggs = []
    for h in range(NSPLIT):
        gt, gxd = _gather_kernel(t, xd, sidx[h], didx[h])
        msgs = _edge_mlp(h, gt, gxd, edge_attr, ea_w1, r(ea_b1), ea_w2,
                         r(ea_b2), te_w1, r(te_b1), te_w2, r(te_b2))
        agg0, agg1 = _scatter_kernel(msgs, didx[h], zeros)
        aggs += [agg0, agg1]

    return _final(x, aggs, pd_w, r(pd_b), pe_w, r(pe_b), phi_w, r(phi_b))


# speedup vs baseline: 4.0497x; 1.1199x over previous
import functools

import jax
import jax.numpy as jnp
from jax import lax
from jax.experimental import pallas as pl
from jax.experimental.pallas import tpu as pltpu
from jax.experimental.pallas import tpu_sc as plsc

N = 10000
E = 320000
D = 128
DE = 16
H = 128

NC = 2
NS = 16
NW = NC * NS
CHUNK = 80
NBUF = 4
ES0 = 163840
ES1 = E - ES0
NP = 10240
RPT = NP // NS

_F32 = jnp.float32
_BF16 = jnp.bfloat16


def _dot(a, b):
    return jnp.dot(a, b, preferred_element_type=_F32)


def _bits16(v):
    return jax.lax.bitcast_convert_type(v.astype(_BF16), jnp.uint16).astype(jnp.uint32)


def _pack2(lo, hi):
    return jax.lax.bitcast_convert_type(_bits16(lo) | (_bits16(hi) << 16), _F32)


def _unpack_lo(w):
    return jax.lax.bitcast_convert_type(w << 16, _F32)


def _unpack_hi(w):
    return jax.lax.bitcast_convert_type(w & jnp.uint32(0xFFFF0000), _F32)


def _node_pre_body(x_ref, sw_ref, sb_ref, dw_ref, db_ref, t_ref, xd_ref):
    xb = x_ref[...]
    xs = _dot(xb, sw_ref[...]) + sb_ref[...]
    xd = _dot(xb, dw_ref[...]) + db_ref[...]
    t_ref[...] = _pack2(xb, xs)
    xd_ref[...] = xd


def _node_pre(x, src_w, src_b, dst_w, dst_b):
    BN = 2000
    return pl.pallas_call(
        _node_pre_body,
        grid=(N // BN,),
        in_specs=[
            pl.BlockSpec((BN, D), lambda i: (i, 0)),
            pl.BlockSpec((D, H), lambda i: (0, 0)),
            pl.BlockSpec((1, H), lambda i: (0, 0)),
            pl.BlockSpec((D, H), lambda i: (0, 0)),
            pl.BlockSpec((1, H), lambda i: (0, 0)),
        ],
        out_specs=[
            pl.BlockSpec((BN, D), lambda i: (i, 0)),
            pl.BlockSpec((BN, H), lambda i: (i, 0)),
        ],
        out_shape=[
            jax.ShapeDtypeStruct((N, D), _F32),
            jax.ShapeDtypeStruct((N, H), _F32),
        ],
    )(x, src_w, src_b, dst_w, dst_b)


_MESH = plsc.VectorSubcoreMesh(core_axis_name="c", subcore_axis_name="s")


def _ring_schedule(nchunk, wait_emit, issue):
    for b in range(min(NBUF, nchunk)):
        issue(b, b)
    full = max(0, (nchunk - NBUF) // NBUF)

    def body(jj, carry):
        for b in range(NBUF):
            j = jj * NBUF + b
            wait_emit(j, b)
            issue(j + NBUF, b)
        return carry

    lax.fori_loop(0, full, body, 0)
    for j in range(full * NBUF, nchunk):
        wait_emit(j, j % NBUF)
        if j + NBUF < nchunk:
            issue(j + NBUF, j % NBUF)


def _make_gather(ew, nchunk):
    @functools.partial(
        pl.kernel,
        mesh=_MESH,
        out_type=(
            jax.ShapeDtypeStruct((ew * NW, D), _F32),
            jax.ShapeDtypeStruct((ew * NW, D), _F32),
        ),
        scratch_types=[
            pltpu.VMEM((nchunk, CHUNK), jnp.int32),
            pltpu.VMEM((nchunk, CHUNK), jnp.int32),
            pltpu.VMEM((NBUF, CHUNK, D), _F32),
            pltpu.VMEM((NBUF, CHUNK, D), _F32),
        ] + [pltpu.SemaphoreType.DMA] * NBUF,
    )
    def gather_k(t_hbm, xd_hbm, sidx_hbm, didx_hbm, gt_hbm, gxd_hbm,
                 sidx_v, didx_v, rt, rd, *sems):
        c = lax.axis_index("c")
        s = lax.axis_index("s")
        wid = s * NC + c
        base = wid * ew
        pltpu.sync_copy(sidx_hbm.at[wid], sidx_v)
        pltpu.sync_copy(didx_hbm.at[wid], didx_v)

        def issue(j, b):
            pltpu.async_copy(t_hbm.at[sidx_v.at[j]], rt.at[b], sems[b])
            pltpu.async_copy(xd_hbm.at[didx_v.at[j]], rd.at[b], sems[b])

        def wait_emit(j, b):
            pltpu.make_async_copy(t_hbm.at[sidx_v.at[j]], rt.at[b], sems[b]).wait()
            pltpu.make_async_copy(xd_hbm.at[didx_v.at[j]], rd.at[b], sems[b]).wait()
            off = base + j * CHUNK
            pltpu.sync_copy(rt.at[b], gt_hbm.at[pl.ds(off, CHUNK)])
            pltpu.sync_copy(rd.at[b], gxd_hbm.at[pl.ds(off, CHUNK)])

        _ring_schedule(nchunk, wait_emit, issue)

    return gather_k


_GATHER = (_make_gather(ES0 // NW, ES0 // NW // CHUNK),
           _make_gather(ES1 // NW, ES1 // NW // CHUNK))


def _edge_mlp_body(gt_ref, gd_ref, ea_ref, w1_ref, b1_ref, w2_ref, b2_ref,
                   tw1_ref, tb1_ref, tw2_ref, tb2_ref, msg_ref):
    wt = jax.lax.bitcast_convert_type(gt_ref[...], jnp.uint32)
    gx = _unpack_lo(wt)
    gs = _unpack_hi(wt)
    c1 = jnp.maximum(_dot(ea_ref[...], w1_ref[...]) + b1_ref[...], 0.0)
    ec = _dot(c1, w2_ref[...]) + b2_ref[...]
    s = ec + gs + gd_ref[...]
    h = jnp.maximum(_dot(jnp.maximum(s, 0.0), tw1_ref[...]) + tb1_ref[...], 0.0)
    e = _dot(h, tw2_ref[...]) + tb2_ref[...]
    msg_ref[...] = gx * e


def _edge_mlp(ne, off_blocks, gt, gxd, edge_attr, ea_w1, ea_b1, ea_w2, ea_b2,
              te_w1, te_b1, te_w2, te_b2):
    BE = 1280
    full = lambda i: (0, 0)
    return pl.pallas_call(
        _edge_mlp_body,
        grid=(ne // BE,),
        in_specs=[
            pl.BlockSpec((BE, D), lambda i: (i, 0)),
            pl.BlockSpec((BE, D), lambda i: (i, 0)),
            pl.BlockSpec((BE, DE), lambda i: (i + off_blocks, 0)),
            pl.BlockSpec((DE, H), full),
            pl.BlockSpec((1, H), full),
            pl.BlockSpec((H, H), full),
            pl.BlockSpec((1, H), full),
            pl.BlockSpec((H, H), full),
            pl.BlockSpec((1, H), full),
            pl.BlockSpec((H, D), full),
            pl.BlockSpec((1, D), full),
        ],
        out_specs=pl.BlockSpec((BE, D), lambda i: (i, 0)),
        out_shape=jax.ShapeDtypeStruct((ne, D), _F32),
    )(gt, gxd, edge_attr, ea_w1, ea_b1, ea_w2, ea_b2,
      te_w1, te_b1, te_w2, te_b2)


def _make_scatter(ew, nchunk):
    @functools.partial(
        pl.kernel,
        mesh=_MESH,
        out_type=(
            jax.ShapeDtypeStruct((NP, D), _F32),
            jax.ShapeDtypeStruct((NP, D), _F32),
        ),
        scratch_types=[
            pltpu.VMEM((nchunk, CHUNK), jnp.int32),
            pltpu.VMEM((NBUF, CHUNK, D), _F32),
            pltpu.VMEM_SHARED((NP, D), _F32),
        ] + [pltpu.SemaphoreType.DMA] * NBUF,
    )
    def scatter_k(msgs_hbm, didx_hbm, zeros_hbm, agg0_hbm, agg1_hbm,
                  didx_v, rv, acc_sh, *sems):
        c = lax.axis_index("c")
        s = lax.axis_index("s")
        wid = s * NC + c
        base = wid * ew
        pltpu.sync_copy(zeros_hbm.at[pl.ds(s * RPT, RPT)],
                        acc_sh.at[pl.ds(s * RPT, RPT)])
        pltpu.sync_copy(didx_hbm.at[wid], didx_v)
        plsc.subcore_barrier()

        def issue(j, b):
            pltpu.async_copy(msgs_hbm.at[pl.ds(base + j * CHUNK, CHUNK)],
                             rv.at[b], sems[b])

        def wait_emit(j, b):
            pltpu.make_async_copy(msgs_hbm.at[pl.ds(base + j * CHUNK, CHUNK)],
                                  rv.at[b], sems[b]).wait()
            pltpu.sync_copy(rv.at[b], acc_sh.at[didx_v.at[j]], add=True)

        _ring_schedule(nchunk, wait_emit, issue)
        plsc.subcore_barrier()

        @pl.when(c == 0)
        def _():
            pltpu.sync_copy(acc_sh.at[pl.ds(s * RPT, RPT)],
                            agg0_hbm.at[pl.ds(s * RPT, RPT)])

        @pl.when(c == 1)
        def _():
            pltpu.sync_copy(acc_sh.at[pl.ds(s * RPT, RPT)],
                            agg1_hbm.at[pl.ds(s * RPT, RPT)])

    return scatter_k


_SCATTER = (_make_scatter(ES0 // NW, ES0 // NW // CHUNK),
            _make_scatter(ES1 // NW, ES1 // NW // CHUNK))


def _final_body(x_ref, p0_ref, p1_ref, p2_ref, p3_ref, pdw_ref, pdb_ref,
                pew_ref, peb_ref, phw_ref, phb_ref, out_ref):
    agg = (p0_ref[...] + p1_ref[...]) + (p2_ref[...] + p3_ref[...])
    pre = _dot(x_ref[...], pdw_ref[...]) + pdb_ref[...]
    pre = pre + _dot(agg, pew_ref[...]) + peb_ref[...]
    out_ref[...] = _dot(jnp.maximum(pre, 0.0), phw_ref[...]) + phb_ref[...]


def _final(x, aggs, pd_w, pd_b, pe_w, pe_b, phi_w, phi_b):
    BN = 2000
    full = lambda i: (0, 0)
    blk = lambda i: (i, 0)
    return pl.pallas_call(
        _final_body,
        grid=(N // BN,),
        in_specs=[
            pl.BlockSpec((BN, D), blk),
            pl.BlockSpec((BN, D), blk),
            pl.BlockSpec((BN, D), blk),
            pl.BlockSpec((BN, D), blk),
            pl.BlockSpec((BN, D), blk),
            pl.BlockSpec((D, H), full),
            pl.BlockSpec((1, H), full),
            pl.BlockSpec((D, H), full),
            pl.BlockSpec((1, H), full),
            pl.BlockSpec((H, D), full),
            pl.BlockSpec((1, D), full),
        ],
        out_specs=pl.BlockSpec((BN, D), blk),
        out_shape=jax.ShapeDtypeStruct((N, D), _F32),
    )(x, *aggs, pd_w, pd_b, pe_w, pe_b, phi_w, phi_b)


def kernel(x, edge_index, edge_attr, ea_w1, ea_b1, ea_w2, ea_b2,
           src_w, src_b, dst_w, dst_b, te_w1, te_b1, te_w2, te_b2,
           pd_w, pd_b, pe_w, pe_b, phi_w, phi_b):
    src = edge_index[0]
    dst = edge_index[1]
    splits = (ES0, ES1)
    bounds = (0, ES0, E)

    r = lambda b: b.reshape(1, -1)

    t, xd = _node_pre(x, src_w, r(src_b), dst_w, r(dst_b))
    zeros = jnp.zeros((NP, D), _F32)

    aggs = []
    for h in range(2):
        es = splits[h]
        ew = es // NW
        nch = ew // CHUNK
        sidx = src[bounds[h]:bounds[h + 1]].reshape(NW, nch, CHUNK)
        didx = dst[bounds[h]:bounds[h + 1]].reshape(NW, nch, CHUNK)
        gt, gxd = _GATHER[h](t, xd, sidx, didx)
        msgs = _edge_mlp(es, bounds[h] // 1280, gt, gxd, edge_attr,
                         ea_w1, r(ea_b1), ea_w2, r(ea_b2),
                         te_w1, r(te_b1), te_w2, r(te_b2))
        agg0, agg1 = _SCATTER[h](msgs, didx, zeros)
        aggs += [agg0, agg1]

    return _final(x, aggs, pd_w, r(pd_b), pe_w, r(pe_b), phi_w, r(phi_b))
